# baseline (speedup 1.0000x reference)
import functools

import jax
import jax.numpy as jnp
from jax import lax
from jax.experimental import pallas as pl
from jax.experimental.pallas import tpu as pltpu
from jax.experimental.pallas import tpu_sc as plsc

_K = 8192
_D = 32
_T = 1024
_KC = 2048
_RB = 64


def _vq_tile_kernel(xs_ref, cb_ref, s2_ref, codes_ref, part_ref):
    xv = xs_ref[...]
    xs = xv * 2.0
    s1 = jnp.sum(xv * xv, axis=1, keepdims=True)
    s2 = s2_ref[...]

    num_chunks = _K // _KC
    mm2 = []
    for c in range(num_chunks):
        cb_c = cb_ref[c * _KC:(c + 1) * _KC, :]
        mm2.append(lax.dot_general(
            xs, cb_c, (((1,), (1,)), ((), ())),
            preferred_element_type=jnp.float32,
        ))

    num_h = _K // 128
    h_per_chunk = _KC // 128
    part = None
    for rb in range(_T // _RB):
        r0 = rb * _RB
        s1_r = s1[r0:r0 + _RB, :]
        m_acc = None
        h_acc = None
        for h in range(num_h):
            c, j = divmod(h, h_per_chunk)
            sl = mm2[c][r0:r0 + _RB, j * 128:(j + 1) * 128]
            s2_h = s2[:, h * 128:(h + 1) * 128]
            dj = (s1_r + s2_h) - sl
            if h == 0:
                m_acc = dj
                h_acc = jnp.zeros((_RB, 128), jnp.float32)
            else:
                mask = dj < m_acc
                h_acc = jnp.where(mask, jnp.float32(h), h_acc)
                m_acc = jnp.minimum(m_acc, dj)
        gmin = jnp.min(m_acc, axis=1, keepdims=True)
        liota = lax.broadcasted_iota(jnp.int32, (_RB, 128), 1).astype(jnp.float32)
        code_f = jnp.min(
            jnp.where(m_acc == gmin, h_acc * 128.0 + liota, jnp.float32(_K)),
            axis=1, keepdims=True,
        )
        codes_ref[r0:r0 + _RB, :] = code_f.astype(jnp.int32)
        psum = jnp.sum(gmin, axis=(0, 1), keepdims=True)
        part = psum if part is None else part + psum
    part_ref[0] = part


def _make_sc_gather(V, DP, B):
    info = plsc.get_sparse_core_info()
    NC, NS = info.num_cores, info.num_subcores
    NW = NC * NS
    b_per_w = B // NW
    CH = 256
    n_ch = b_per_w // CH
    mesh = plsc.VectorSubcoreMesh(core_axis_name="c", subcore_axis_name="s")

    @functools.partial(
        pl.kernel, mesh=mesh,
        out_type=jax.ShapeDtypeStruct((B, DP), jnp.float32),
        scratch_types=[
            pltpu.VMEM((b_per_w,), jnp.int32),
            pltpu.VMEM((CH, DP), jnp.float32),
            pltpu.SemaphoreType.DMA,
        ],
    )
    def sc_gather(table_hbm, idx_hbm, out_hbm, idx_v, rows_v, sem):
        wid = lax.axis_index("s") * NC + lax.axis_index("c")
        base = wid * b_per_w
        pltpu.sync_copy(idx_hbm.at[pl.ds(base, b_per_w)], idx_v)
        for cc in range(n_ch):
            pltpu.async_copy(
                table_hbm.at[idx_v.at[pl.ds(cc * CH, CH)]], rows_v, sem
            ).wait()
            pltpu.sync_copy(rows_v, out_hbm.at[pl.ds(base + cc * CH, CH)])

    return sc_gather


def kernel(x, codebook):
    B, S, D = x.shape
    N = B * S
    s2 = jnp.sum(codebook ** 2, axis=-1)

    x2 = x.reshape(N, D)
    s2_2 = s2.reshape(1, _K)
    grid = (N // _T,)

    NH = N // 2
    gridh = (NH // _T,)

    def tc_half(xh):
        return pl.pallas_call(
            _vq_tile_kernel,
            grid=gridh,
            in_specs=[
                pl.BlockSpec((_T, D), lambda i: (i, 0)),
                pl.BlockSpec((_K, D), lambda i: (0, 0)),
                pl.BlockSpec((1, _K), lambda i: (0, 0)),
            ],
            out_specs=[
                pl.BlockSpec((_T, 1), lambda i: (i, 0)),
                pl.BlockSpec((1, 1, 1), lambda i: (i, 0, 0)),
            ],
            out_shape=[
                jax.ShapeDtypeStruct((NH, 1), jnp.int32),
                jax.ShapeDtypeStruct((gridh[0], 1, 1), jnp.float32),
            ],
            compiler_params=pltpu.CompilerParams(
                dimension_semantics=("parallel",),
            ),
        )(xh, codebook, s2_2)

    cb_pad = jnp.pad(codebook, ((0, 0), (0, 128 - D)))
    sc_gather = _make_sc_gather(_K, 128, NH)
    codes_a, parts_a = tc_half(x2[:NH])
    q_pad_a = sc_gather(cb_pad, codes_a.reshape(NH))
    codes_b, parts_b = tc_half(x2[NH:])
    q_pad_b = sc_gather(cb_pad, codes_b.reshape(NH))

    codes = jnp.concatenate([codes_a, codes_b], axis=0).reshape(B, S)
    q2 = jnp.concatenate([q_pad_a[:, :D], q_pad_b[:, :D]], axis=0)
    quantized_st = q2.reshape(B, S, D)
    loss = 2.0 * ((jnp.sum(parts_a) + jnp.sum(parts_b)) / (N * D))
    return (quantized_st, codes, loss)


# speedup vs baseline: 1.0738x; 1.0738x over previous
import functools

import jax
import jax.numpy as jnp
from jax import lax
from jax.experimental import pallas as pl
from jax.experimental.pallas import tpu as pltpu
from jax.experimental.pallas import tpu_sc as plsc

_K = 8192
_D = 32
_T = 1024
_KC = 2048
_RB = 64


def _vq_tile_kernel(xs_ref, cb_ref, s2_ref, codes_ref, part_ref):
    xv = xs_ref[...]
    xs = xv * 2.0
    s1 = jnp.sum(xv * xv, axis=1, keepdims=True)
    s2 = s2_ref[...]

    num_chunks = _K // _KC
    mm2 = []
    for c in range(num_chunks):
        cb_c = cb_ref[c * _KC:(c + 1) * _KC, :]
        mm2.append(lax.dot_general(
            xs, cb_c, (((1,), (1,)), ((), ())),
            preferred_element_type=jnp.float32,
        ))

    num_h = _K // 128
    h_per_chunk = _KC // 128
    part = None
    for rb in range(_T // _RB):
        r0 = rb * _RB
        s1_r = s1[r0:r0 + _RB, :]
        m_acc = None
        h_acc = None
        for h in range(num_h):
            c, j = divmod(h, h_per_chunk)
            sl = mm2[c][r0:r0 + _RB, j * 128:(j + 1) * 128]
            s2_h = s2[:, h * 128:(h + 1) * 128]
            dj = (s1_r + s2_h) - sl
            if h == 0:
                m_acc = dj
                h_acc = jnp.zeros((_RB, 128), jnp.float32)
            else:
                mask = dj < m_acc
                h_acc = jnp.where(mask, jnp.float32(h), h_acc)
                m_acc = jnp.minimum(m_acc, dj)
        gmin = jnp.min(m_acc, axis=1, keepdims=True)
        liota = lax.broadcasted_iota(jnp.int32, (_RB, 128), 1).astype(jnp.float32)
        code_f = jnp.min(
            jnp.where(m_acc == gmin, h_acc * 128.0 + liota, jnp.float32(_K)),
            axis=1, keepdims=True,
        )
        codes_ref[r0:r0 + _RB, :] = code_f.astype(jnp.int32)
        psum = jnp.sum(gmin, axis=(0, 1), keepdims=True)
        part = psum if part is None else part + psum
    part_ref[0] = part


def _make_sc_gather(V, DP, B):
    info = plsc.get_sparse_core_info()
    NC, NS = info.num_cores, info.num_subcores
    NW = NC * NS
    b_per_w = B // NW
    CH = 256
    n_ch = b_per_w // CH
    mesh = plsc.VectorSubcoreMesh(core_axis_name="c", subcore_axis_name="s")

    @functools.partial(
        pl.kernel, mesh=mesh,
        out_type=jax.ShapeDtypeStruct((B, DP), jnp.float32),
        scratch_types=[
            pltpu.VMEM((b_per_w,), jnp.int32),
            pltpu.VMEM((CH, DP), jnp.float32),
            pltpu.VMEM((CH, DP), jnp.float32),
            pltpu.SemaphoreType.DMA,
            pltpu.SemaphoreType.DMA,
            pltpu.SemaphoreType.DMA,
            pltpu.SemaphoreType.DMA,
        ],
    )
    def sc_gather(table_hbm, idx_hbm, out_hbm, idx_v, rows0, rows1,
                  g0, g1, w0, w1):
        wid = lax.axis_index("s") * NC + lax.axis_index("c")
        base = wid * b_per_w
        pltpu.sync_copy(idx_hbm.at[pl.ds(base, b_per_w)], idx_v)
        bufs = (rows0, rows1)
        gsems = (g0, g1)
        wsems = (w0, w1)
        gathers = [None] * n_ch
        writes = [None] * n_ch
        gathers[0] = pltpu.async_copy(
            table_hbm.at[idx_v.at[pl.ds(0, CH)]], bufs[0], gsems[0])
        for cc in range(n_ch):
            gathers[cc].wait()
            writes[cc] = pltpu.async_copy(
                bufs[cc % 2], out_hbm.at[pl.ds(base + cc * CH, CH)],
                wsems[cc % 2])
            if cc + 1 < n_ch:
                if cc >= 1:
                    writes[cc - 1].wait()
                gathers[cc + 1] = pltpu.async_copy(
                    table_hbm.at[idx_v.at[pl.ds((cc + 1) * CH, CH)]],
                    bufs[(cc + 1) % 2], gsems[(cc + 1) % 2])
        writes[n_ch - 2].wait()
        writes[n_ch - 1].wait()

    return sc_gather


def kernel(x, codebook):
    B, S, D = x.shape
    N = B * S
    s2 = jnp.sum(codebook ** 2, axis=-1)

    x2 = x.reshape(N, D)
    s2_2 = s2.reshape(1, _K)
    grid = (N // _T,)

    codes2, parts = pl.pallas_call(
        _vq_tile_kernel,
        grid=grid,
        in_specs=[
            pl.BlockSpec((_T, D), lambda i: (i, 0)),
            pl.BlockSpec((_K, D), lambda i: (0, 0)),
            pl.BlockSpec((1, _K), lambda i: (0, 0)),
        ],
        out_specs=[
            pl.BlockSpec((_T, 1), lambda i: (i, 0)),
            pl.BlockSpec((1, 1, 1), lambda i: (i, 0, 0)),
        ],
        out_shape=[
            jax.ShapeDtypeStruct((N, 1), jnp.int32),
            jax.ShapeDtypeStruct((grid[0], 1, 1), jnp.float32),
        ],
        compiler_params=pltpu.CompilerParams(
            dimension_semantics=("parallel",),
        ),
    )(x2, codebook, s2_2)

    codes_flat = codes2.reshape(N)
    cb_pad = jnp.pad(codebook, ((0, 0), (0, 128 - D)))
    q_pad = _make_sc_gather(_K, 128, N)(cb_pad, codes_flat)
    q2 = q_pad[:, :D]

    codes = codes2.reshape(B, S)
    quantized_st = q2.reshape(B, S, D)
    loss = 2.0 * (jnp.sum(parts) / (N * D))
    return (quantized_st, codes, loss)
